# baseline (speedup 1.0000x reference)
import functools

import jax
import jax.numpy as jnp
from jax import lax
from jax.experimental import pallas as pl
from jax.experimental.pallas import tpu as pltpu
from jax.experimental.pallas import tpu_sc as plsc

D_MODEL = 64
SCALE = 8.0
NC, NS = 2, 16
NW = NC * NS
NBUF = 4
CHUNKS = (104, 96)


@functools.lru_cache(maxsize=None)
def _make_kernel(n_seq, seq_len):
    s_per_w = n_seq // NW
    n_super = s_per_w // 2
    offs = (0, CHUNKS[0])
    mesh = plsc.VectorSubcoreMesh(
        core_axis_name="c", subcore_axis_name="s", num_cores=NC, num_subcores=NS
    )

    def clen(b):
        return CHUNKS[b & 1]

    def coff(b):
        return offs[b & 1]

    @functools.partial(
        pl.kernel,
        out_type=jax.ShapeDtypeStruct((n_seq, seq_len, D_MODEL), jnp.float32),
        mesh=mesh,
        scratch_types=[
            pltpu.VMEM((s_per_w, seq_len), jnp.int32),
            [pltpu.VMEM((clen(b), D_MODEL), jnp.float32) for b in range(NBUF)],
            [pltpu.VMEM((clen(b), D_MODEL), jnp.float32) for b in range(NBUF)],
            [pltpu.SemaphoreType.DMA] * NBUF,
            [pltpu.SemaphoreType.DMA] * NBUF,
        ],
        compiler_params=pltpu.CompilerParams(use_tc_tiling_on_sc=False),
    )
    def emb(x_hbm, table_hbm, out_hbm, idx_v, gbuf, wbuf, gsem, wsem):
        wid = lax.axis_index("s") * NC + lax.axis_index("c")
        seq0 = wid * s_per_w
        pltpu.sync_copy(x_hbm.at[pl.ds(seq0, s_per_w)], idx_v)

        def gather_start(b, s_local):
            pltpu.async_copy(
                table_hbm.at[idx_v.at[s_local, pl.ds(coff(b), clen(b))]],
                gbuf[b],
                gsem[b],
            )

        def gather_wait(b, s_local):
            pltpu.make_async_copy(
                table_hbm.at[idx_v.at[s_local, pl.ds(coff(b), clen(b))]],
                gbuf[b],
                gsem[b],
            ).wait()

        def write_start(b, s_local):
            pltpu.async_copy(
                wbuf[b],
                out_hbm.at[seq0 + s_local, pl.ds(coff(b), clen(b))],
                wsem[b],
            )

        def write_wait(b):
            pltpu.make_async_copy(
                wbuf[b],
                out_hbm.at[seq0, pl.ds(coff(b), clen(b))],
                wsem[b],
            ).wait()

        for b in range(NBUF):
            gather_start(b, b >> 1)

        def superstep(t, carry):
            for b in range(NBUF):
                s_local = 2 * t + (b >> 1)
                gather_wait(b, s_local)

                @pl.when(t != 0)
                def _():
                    write_wait(b)

                gb, wb = gbuf[b], wbuf[b]

                @plsc.parallel_loop(0, clen(b), step=1, unroll=8)
                def _(r):
                    for kk in range(D_MODEL // 16):
                        sl = pl.ds(kk * 16, 16)
                        wb[r, sl] = gb[r, sl] * SCALE

                write_start(b, s_local)

                @pl.when(t != n_super - 1)
                def _():
                    gather_start(b, 2 * t + 2 + (b >> 1))

            return carry

        lax.fori_loop(0, n_super, superstep, 0)

        for b in range(NBUF):
            write_wait(b)

    return emb


def kernel(x, table):
    return _make_kernel(x.shape[0], x.shape[1])(x.astype(jnp.int32), table)
